# baseline (speedup 1.0000x reference)
import functools

import jax
import jax.numpy as jnp
from jax.experimental import pallas as pl
from jax.experimental.pallas import tpu as pltpu

_CHUNK = 50


def _embed_ln_kernel(g_ref, s_ref, pos_ref, mod_ref, w_ref, b_ref, out_ref):
    w = w_ref[:, :]
    b = b_ref[:, :]
    d = w.shape[-1]
    bb = out_ref.shape[0]
    avg = jnp.full((d, d), 1.0 / d, dtype=jnp.float32)

    def body(x, bias, k):
        x = (x + bias[None, :, :]).reshape(bb * _CHUNK, d)
        mu = jax.lax.dot(x, avg, precision=jax.lax.Precision.HIGHEST)
        msq = jax.lax.dot(x * x, avg, precision=jax.lax.Precision.HIGHEST)
        var = msq - mu * mu
        xn = (x - mu) * jax.lax.rsqrt(var + 1e-05)
        out_ref[:, k, :, :] = (xn * w + b).reshape(bb, _CHUNK, d)

    body(g_ref[:, 0, :, :], pos_ref[0, :, :] + mod_ref[0, :, :], 0)
    for k in range(1, 5):
        body(s_ref[:, k - 1, :, :], pos_ref[k, :, :] + mod_ref[1, :, :], k)


@functools.partial(jax.jit, static_argnames=())
def kernel(smiles_feats, graph_feats, pos_table, mod_table, ln_weight, ln_bias):
    b_dim, sg, d = graph_feats.shape
    ss = smiles_feats.shape[1]
    total = sg + ss
    n_chunks = total // _CHUNK
    bb = 64

    gf = graph_feats.reshape(b_dim, sg // _CHUNK, _CHUNK, d)
    sf = smiles_feats.reshape(b_dim, ss // _CHUNK, _CHUNK, d)
    pos = pos_table[:total].reshape(n_chunks, _CHUNK, d)
    mod = mod_table.reshape(2, 1, d)
    w = ln_weight.reshape(1, d)
    bias = ln_bias.reshape(1, d)

    grid = (b_dim // bb,)
    out = pl.pallas_call(
        _embed_ln_kernel,
        grid=grid,
        in_specs=[
            pl.BlockSpec((bb, 1, _CHUNK, d), lambda i: (i, 0, 0, 0)),
            pl.BlockSpec((bb, 4, _CHUNK, d), lambda i: (i, 0, 0, 0)),
            pl.BlockSpec((n_chunks, _CHUNK, d), lambda i: (0, 0, 0)),
            pl.BlockSpec((2, 1, d), lambda i: (0, 0, 0)),
            pl.BlockSpec((1, d), lambda i: (0, 0)),
            pl.BlockSpec((1, d), lambda i: (0, 0)),
        ],
        out_specs=pl.BlockSpec((bb, n_chunks, _CHUNK, d), lambda i: (i, 0, 0, 0)),
        out_shape=jax.ShapeDtypeStruct((b_dim, n_chunks, _CHUNK, d), jnp.float32),
        compiler_params=pltpu.CompilerParams(
            dimension_semantics=("parallel",),
            vmem_limit_bytes=100 * 1024 * 1024,
        ),
    )(gf, sf, pos, mod, w, bias)
    return out.reshape(b_dim, total, d)


# speedup vs baseline: 1.0444x; 1.0444x over previous
import functools

import jax
import jax.numpy as jnp
from jax import lax
from jax.experimental import pallas as pl
from jax.experimental.pallas import tpu as pltpu
from jax.experimental.pallas import tpu_sc as plsc

_B = 1024
_SG = 50
_SS = 200
_TOT = _SG + _SS
_D = 128
_NW = 32
_BPW = _B // _NW
_GW = _SG * _D
_SW = _SS * _D
_TW = _TOT * _D
_L = 16


def _rsqrt16(v):
    i = lax.bitcast_convert_type(v, jnp.int32)
    i = jnp.int32(0x5F3759DF) - lax.shift_right_logical(i, jnp.int32(1))
    y = lax.bitcast_convert_type(i, jnp.float32)
    half = jnp.float32(0.5) * v
    for _ in range(3):
        y = y * (jnp.float32(1.5) - half * y * y)
    return y


_GATHER_DNUMS = lax.GatherDimensionNumbers(
    offset_dims=(), collapsed_slice_dims=(0,), start_index_map=(0,)
)


def _take16(x, idx):
    return lax.gather(
        x, idx[:, None], _GATHER_DNUMS, slice_sizes=(1,),
        mode=lax.GatherScatterMode.PROMISE_IN_BOUNDS,
    )


def _allsum16(x):
    lanes = lax.iota(jnp.int32, _L)
    for m in (8, 4, 2, 1):
        x = x + _take16(x, lanes ^ m)
    return x


def _ln_tokens(buf, bias, wv, bv):
    inv_d = jnp.float32(1.0 / _D)

    def token(t, carry):
        base = t * _D
        xs = []
        for j in range(_D // _L):
            x = buf[pl.ds(base + j * _L, _L)] + bias[pl.ds(base + j * _L, _L)]
            xs.append(x)
        s = xs[0]
        q = xs[0] * xs[0]
        for j in range(1, _D // _L):
            s = s + xs[j]
            q = q + xs[j] * xs[j]
        mu = _allsum16(s) * inv_d
        msq = _allsum16(q) * inv_d
        var = msq - mu * mu
        rinv = _rsqrt16(var + jnp.float32(1e-05))
        for j in range(_D // _L):
            y = (xs[j] - mu) * rinv * wv[j] + bv[j]
            buf[pl.ds(base + j * _L, _L)] = y
        return carry

    lax.fori_loop(0, _TOT, token, 0)


def _sc_body(graph_hbm, smiles_hbm, pos_hbm, mod_hbm, w_hbm, b_hbm, out_hbm,
             buf0, buf1, bias, wb, sem_in0, sem_in1, sem_out0, sem_out1):
    wid = lax.axis_index("s") * 2 + lax.axis_index("c")
    base_row = wid * _BPW

    pltpu.sync_copy(w_hbm.at[pl.ds(0, _D)], wb.at[pl.ds(0, _D)])
    pltpu.sync_copy(b_hbm.at[pl.ds(0, _D)], wb.at[pl.ds(_D, _D)])
    pltpu.sync_copy(mod_hbm.at[pl.ds(0, 2 * _D)], wb.at[pl.ds(2 * _D, 2 * _D)])
    pltpu.sync_copy(pos_hbm.at[pl.ds(0, _TW)], bias.at[pl.ds(0, _TW)])

    def add_mod(t, carry):
        base = t * _D
        moff = jnp.where(t < _SG, 2 * _D, 3 * _D)
        for j in range(_D // _L):
            sl = pl.ds(base + j * _L, _L)
            bias[sl] = bias[sl] + wb[pl.ds(moff + j * _L, _L)]
        return carry

    lax.fori_loop(0, _TOT, add_mod, 0)

    wv = [wb[pl.ds(j * _L, _L)] for j in range(_D // _L)]
    bv = [wb[pl.ds(_D + j * _L, _L)] for j in range(_D // _L)]

    bufs = (buf0, buf1)
    in_sems = (sem_in0, sem_in1)
    out_sems = (sem_out0, sem_out1)

    def start_in(b, buf, sem):
        row = base_row + b
        hg = pltpu.async_copy(
            graph_hbm.at[pl.ds(row * _GW, _GW)], buf.at[pl.ds(0, _GW)], sem)
        hs = pltpu.async_copy(
            smiles_hbm.at[pl.ds(row * _SW, _SW)], buf.at[pl.ds(_GW, _SW)], sem)
        return (hg, hs)

    def start_out(b, buf, sem):
        row = base_row + b
        return pltpu.async_copy(
            buf.at[pl.ds(0, _TW)], out_hbm.at[pl.ds(row * _TW, _TW)], sem)

    h_in = {0: start_in(0, bufs[0], in_sems[0])}
    h_out = {}
    for b in range(_BPW):
        p = b % 2
        if b + 1 < _BPW:
            if b - 1 >= 0:
                h_out[b - 1].wait()
            h_in[b + 1] = start_in(b + 1, bufs[1 - p], in_sems[1 - p])
        for h in h_in[b]:
            h.wait()
        _ln_tokens(bufs[p], bias, wv, bv)
        h_out[b] = start_out(b, bufs[p], out_sems[p])
    h_out[_BPW - 2].wait()
    h_out[_BPW - 1].wait()


@functools.partial(jax.jit, static_argnames=())
def kernel(smiles_feats, graph_feats, pos_table, mod_table, ln_weight, ln_bias):
    b_dim = graph_feats.shape[0]
    mesh = plsc.VectorSubcoreMesh(core_axis_name="c", subcore_axis_name="s")
    run = pl.kernel(
        _sc_body,
        mesh=mesh,
        out_type=jax.ShapeDtypeStruct((b_dim * _TW,), jnp.float32),
        scratch_types=[
            pltpu.VMEM((_TW,), jnp.float32),
            pltpu.VMEM((_TW,), jnp.float32),
            pltpu.VMEM((_TW,), jnp.float32),
            pltpu.VMEM((4 * _D,), jnp.float32),
            pltpu.SemaphoreType.DMA,
            pltpu.SemaphoreType.DMA,
            pltpu.SemaphoreType.DMA,
            pltpu.SemaphoreType.DMA,
        ],
    )
    out = run(
        graph_feats.reshape(-1),
        smiles_feats.reshape(-1),
        pos_table[:_TOT].reshape(-1),
        mod_table.reshape(-1),
        ln_weight,
        ln_bias,
    )
    return out.reshape(b_dim, _TOT, _D)


# speedup vs baseline: 1.8682x; 1.7888x over previous
import functools

import jax
import jax.numpy as jnp
from jax.experimental import pallas as pl
from jax.experimental.pallas import tpu as pltpu

_CHUNK = 50


def _embed_ln_kernel(g_ref, s_ref, pos_ref, mod_ref, w_ref, b_ref, out_ref):
    w = w_ref[:, :]
    b = b_ref[:, :]

    def body(x, bias, k):
        x = x + bias[None, :, :]
        mu = jnp.mean(x, axis=-1, keepdims=True)
        var = jnp.mean(jnp.square(x - mu), axis=-1, keepdims=True)
        xn = (x - mu) * jax.lax.rsqrt(var + 1e-05)
        out_ref[:, k, :, :] = xn * w + b

    body(g_ref[:, 0, :, :], pos_ref[0, :, :] + mod_ref[0, :, :], 0)
    for k in range(1, 5):
        body(s_ref[:, k - 1, :, :], pos_ref[k, :, :] + mod_ref[1, :, :], k)


@functools.partial(jax.jit, static_argnames=())
def kernel(smiles_feats, graph_feats, pos_table, mod_table, ln_weight, ln_bias):
    b_dim, sg, d = graph_feats.shape
    ss = smiles_feats.shape[1]
    total = sg + ss
    n_chunks = total // _CHUNK
    bb = 64

    gf = graph_feats.reshape(b_dim, sg // _CHUNK, _CHUNK, d)
    sf = smiles_feats.reshape(b_dim, ss // _CHUNK, _CHUNK, d)
    pos = pos_table[:total].reshape(n_chunks, _CHUNK, d)
    mod = mod_table.reshape(2, 1, d)
    w = ln_weight.reshape(1, d)
    bias = ln_bias.reshape(1, d)

    grid = (b_dim // bb,)
    out = pl.pallas_call(
        _embed_ln_kernel,
        grid=grid,
        in_specs=[
            pl.BlockSpec((bb, 1, _CHUNK, d), lambda i: (i, 0, 0, 0)),
            pl.BlockSpec((bb, 4, _CHUNK, d), lambda i: (i, 0, 0, 0)),
            pl.BlockSpec((n_chunks, _CHUNK, d), lambda i: (0, 0, 0)),
            pl.BlockSpec((2, 1, d), lambda i: (0, 0, 0)),
            pl.BlockSpec((1, d), lambda i: (0, 0)),
            pl.BlockSpec((1, d), lambda i: (0, 0)),
        ],
        out_specs=pl.BlockSpec((bb, n_chunks, _CHUNK, d), lambda i: (i, 0, 0, 0)),
        out_shape=jax.ShapeDtypeStruct((b_dim, n_chunks, _CHUNK, d), jnp.float32),
        compiler_params=pltpu.CompilerParams(
            dimension_semantics=("parallel",),
            vmem_limit_bytes=100 * 1024 * 1024,
        ),
    )(gf, sf, pos, mod, w, bias)
    return out.reshape(b_dim, total, d)
